# Initial kernel scaffold; baseline (speedup 1.0000x reference)
import jax
import jax.numpy as jnp
from jax.experimental import pallas as pl


def kernel(query, value, weight):
    raise NotImplementedError("write your pallas kernel here")


# speedup vs baseline: 6.1531x; 6.1531x over previous
import functools

import jax
import jax.numpy as jnp
from jax import lax
from jax.experimental import pallas as pl
from jax.experimental.pallas import tpu as pltpu
from jax.experimental.pallas import tpu_sc as plsc

_NUM_CORES = 2
_NUM_SUBCORES = 16
_LANES = 16
_IDX_CHUNK = 128


def _build_sc_kernel(q_len, v_len, vocab, dim):
  mp = (vocab - 1) // 2
  n_workers = _NUM_CORES * _NUM_SUBCORES
  assert q_len % n_workers == 0
  rows_per_worker = q_len // n_workers
  slice_rows = v_len + rows_per_worker - 1
  slice_pad = -slice_rows % _IDX_CHUNK
  n_chunks = (slice_rows + slice_pad) // _IDX_CHUNK
  assert dim % _LANES == 0

  mesh = plsc.VectorSubcoreMesh(
      core_axis_name="c", subcore_axis_name="s")

  @functools.partial(
      pl.kernel,
      out_type=jax.ShapeDtypeStruct((q_len, v_len, dim), jnp.float32),
      mesh=mesh,
      scratch_types=[
          pltpu.VMEM((n_chunks, _IDX_CHUNK), jnp.int32),
          pltpu.VMEM((slice_rows + slice_pad, dim), jnp.float32),
          pltpu.SemaphoreType.DMA,
      ],
      compiler_params=pltpu.CompilerParams(use_tc_tiling_on_sc=False),
  )
  def body(weight_hbm, out_hbm, idx_v, l_v, sem):
    c = lax.axis_index("c")
    s = lax.axis_index("s")
    wid = s * _NUM_CORES + c
    base = wid * rows_per_worker

    off0 = mp + 1 - rows_per_worker - base
    lanes = lax.iota(jnp.int32, _LANES)
    for ch in range(n_chunks):
      for k in range(_IDX_CHUNK // _LANES):
        m0 = ch * _IDX_CHUNK + k * _LANES
        vals = jnp.clip(lanes + (m0 + off0), 0, 2 * mp)
        idx_v[ch, pl.ds(k * _LANES, _LANES)] = vals

    copies = [
        pltpu.async_copy(
            weight_hbm.at[idx_v.at[ch]],
            l_v.at[pl.ds(ch * _IDX_CHUNK, _IDX_CHUNK)],
            sem)
        for ch in range(n_chunks)
    ]
    for cp in copies:
      cp.wait()

    def row_step(r, _):
      i = base + r
      pltpu.sync_copy(
          l_v.at[pl.ds(rows_per_worker - 1 - r, v_len)],
          out_hbm.at[i])
      return _

    lax.fori_loop(0, rows_per_worker, row_step, None)

  return body


def kernel(query, value, weight):
  q_len = query.shape[1]
  v_len = value.shape[1]
  vocab, dim = weight.shape
  sc = _build_sc_kernel(q_len, v_len, vocab, dim)
  return sc(weight)
